# initial kernel scaffold (unmeasured)
import jax
import jax.numpy as jnp
from jax import lax
from jax.experimental import pallas as pl
from jax.experimental.pallas import tpu as pltpu

N_DEV = 4


def _gemm(x, w):
    M, K = x.shape
    _, N = w.shape
    bm, bn = 512, 2048

    def body(x_ref, w_ref, o_ref):
        o_ref[...] = jnp.dot(
            x_ref[...].astype(jnp.bfloat16),
            w_ref[...].astype(jnp.bfloat16),
            preferred_element_type=jnp.float32,
        )

    return pl.pallas_call(
        body,
        grid=(N // bn, M // bm),
        in_specs=[
            pl.BlockSpec((bm, K), lambda n, m: (m, 0)),
            pl.BlockSpec((K, bn), lambda n, m: (0, n)),
        ],
        out_specs=pl.BlockSpec((bm, bn), lambda n, m: (m, n)),
        out_shape=jax.ShapeDtypeStruct((M, N), jnp.float32),
    )(x, w)


def _rs_silu(partial):
    M, N = partial.shape
    m_out = M // N_DEV
    bn = 1024
    n_blocks = N // bn
    n_hops = N_DEV - 1

    def body(p_ref, o_ref, recv, acc, lbuf, send_sems, recv_sems, lsem, osem, credit):
        my = lax.axis_index("i")
        left = lax.rem(my + N_DEV - 1, N_DEV)
        right = lax.rem(my + 1, N_DEV)

        barrier = pltpu.get_barrier_semaphore()
        for nbr in (left, right):
            pl.semaphore_signal(
                barrier, inc=1, device_id=(nbr,),
                device_id_type=pl.DeviceIdType.MESH,
            )
        pl.semaphore_wait(barrier, 2)

        for b in range(n_blocks):
            cols = pl.ds(b * bn, bn)
            for h in range(n_hops):
                send_chunk = lax.rem(my + N_DEV - 1 - h, N_DEV)
                recv_chunk = lax.rem(my + N_DEV - 2 - h, N_DEV)
                if b * n_hops + h >= n_hops:
                    pl.semaphore_wait(credit, 1)
                src = (
                    p_ref.at[pl.ds(send_chunk * m_out, m_out), cols]
                    if h == 0
                    else acc
                )
                rdma = pltpu.make_async_remote_copy(
                    src_ref=src,
                    dst_ref=recv.at[h],
                    send_sem=send_sems.at[h],
                    recv_sem=recv_sems.at[h],
                    device_id=(right,),
                    device_id_type=pl.DeviceIdType.MESH,
                )
                rdma.start()
                lcopy = pltpu.make_async_copy(
                    p_ref.at[pl.ds(recv_chunk * m_out, m_out), cols], lbuf, lsem
                )
                lcopy.start()
                rdma.wait()
                lcopy.wait()
                s = recv[h] + lbuf[...]
                if h == n_hops - 1:
                    acc[...] = s * jax.nn.sigmoid(s)
                else:
                    acc[...] = s
                pl.semaphore_signal(
                    credit, inc=1, device_id=(left,),
                    device_id_type=pl.DeviceIdType.MESH,
                )
            ocopy = pltpu.make_async_copy(acc, o_ref.at[:, cols], osem)
            ocopy.start()
            ocopy.wait()
        pl.semaphore_wait(credit, n_hops)

    return pl.pallas_call(
        body,
        in_specs=[pl.BlockSpec(memory_space=pltpu.ANY)],
        out_specs=pl.BlockSpec(memory_space=pltpu.ANY),
        out_shape=jax.ShapeDtypeStruct((m_out, N), jnp.float32),
        scratch_shapes=[
            pltpu.VMEM((n_hops, m_out, bn), jnp.float32),
            pltpu.VMEM((m_out, bn), jnp.float32),
            pltpu.VMEM((m_out, bn), jnp.float32),
            pltpu.SemaphoreType.DMA((n_hops,)),
            pltpu.SemaphoreType.DMA((n_hops,)),
            pltpu.SemaphoreType.DMA,
            pltpu.SemaphoreType.DMA,
            pltpu.SemaphoreType.REGULAR,
        ],
        compiler_params=pltpu.CompilerParams(collective_id=0),
    )(partial)


def kernel(x, w_mat):
    partial = _gemm(x, w_mat)
    return _rs_silu(partial)


# baseline (device time: 1321405 ns/iter reference)
import jax
import jax.numpy as jnp
from jax import lax
from jax.experimental import pallas as pl
from jax.experimental.pallas import tpu as pltpu

N_DEV = 4


def _gemm(x, w):
    M, K = x.shape
    _, N = w.shape
    bm, bn = 512, 2048

    def body(x_ref, w_ref, o_ref):
        o_ref[...] = jnp.dot(
            x_ref[...].astype(jnp.bfloat16),
            w_ref[...].astype(jnp.bfloat16),
            preferred_element_type=jnp.float32,
        )

    return pl.pallas_call(
        body,
        grid=(N // bn, M // bm),
        in_specs=[
            pl.BlockSpec((bm, K), lambda n, m: (m, 0)),
            pl.BlockSpec((K, bn), lambda n, m: (0, n)),
        ],
        out_specs=pl.BlockSpec((bm, bn), lambda n, m: (m, n)),
        out_shape=jax.ShapeDtypeStruct((M, N), jnp.float32),
        compiler_params=pltpu.CompilerParams(vmem_limit_bytes=60 * 1024 * 1024),
    )(x, w)


def _rs_silu(partial):
    M, N = partial.shape
    m_out = M // N_DEV
    bn = 1024
    n_blocks = N // bn
    n_hops = N_DEV - 1

    def body(p_ref, o_ref, recv, acc, lbuf, send_sems, recv_sems, lsem, osem, credit):
        my = lax.axis_index("i")
        left = lax.rem(my + N_DEV - 1, N_DEV)
        right = lax.rem(my + 1, N_DEV)

        barrier = pltpu.get_barrier_semaphore()
        for nbr in (left, right):
            pl.semaphore_signal(
                barrier, inc=1, device_id=(nbr,),
                device_id_type=pl.DeviceIdType.MESH,
            )
        pl.semaphore_wait(barrier, 2)

        for b in range(n_blocks):
            cols = pl.ds(b * bn, bn)
            for h in range(n_hops):
                send_chunk = lax.rem(my + N_DEV - 1 - h, N_DEV)
                recv_chunk = lax.rem(my + N_DEV - 2 - h, N_DEV)
                if b * n_hops + h >= n_hops:
                    pl.semaphore_wait(credit, 1)
                src = (
                    p_ref.at[pl.ds(send_chunk * m_out, m_out), cols]
                    if h == 0
                    else acc
                )
                rdma = pltpu.make_async_remote_copy(
                    src_ref=src,
                    dst_ref=recv.at[h],
                    send_sem=send_sems.at[h],
                    recv_sem=recv_sems.at[h],
                    device_id=(right,),
                    device_id_type=pl.DeviceIdType.MESH,
                )
                rdma.start()
                lcopy = pltpu.make_async_copy(
                    p_ref.at[pl.ds(recv_chunk * m_out, m_out), cols], lbuf, lsem
                )
                lcopy.start()
                rdma.wait()
                lcopy.wait()
                s = recv[h] + lbuf[...]
                if h == n_hops - 1:
                    acc[...] = s * jax.nn.sigmoid(s)
                else:
                    acc[...] = s
                pl.semaphore_signal(
                    credit, inc=1, device_id=(left,),
                    device_id_type=pl.DeviceIdType.MESH,
                )
            ocopy = pltpu.make_async_copy(acc, o_ref.at[:, cols], osem)
            ocopy.start()
            ocopy.wait()
        pl.semaphore_wait(credit, n_hops)

    return pl.pallas_call(
        body,
        in_specs=[pl.BlockSpec(memory_space=pl.ANY)],
        out_specs=pl.BlockSpec(memory_space=pl.ANY),
        out_shape=jax.ShapeDtypeStruct((m_out, N), jnp.float32),
        scratch_shapes=[
            pltpu.VMEM((n_hops, m_out, bn), jnp.float32),
            pltpu.VMEM((m_out, bn), jnp.float32),
            pltpu.VMEM((m_out, bn), jnp.float32),
            pltpu.SemaphoreType.DMA((n_hops,)),
            pltpu.SemaphoreType.DMA((n_hops,)),
            pltpu.SemaphoreType.DMA,
            pltpu.SemaphoreType.DMA,
            pltpu.SemaphoreType.REGULAR,
        ],
        compiler_params=pltpu.CompilerParams(
            collective_id=0, vmem_limit_bytes=60 * 1024 * 1024
        ),
    )(partial)


def kernel(x, w_mat):
    partial = _gemm(x, w_mat)
    return _rs_silu(partial)


# device time: 782335 ns/iter; 1.6891x vs baseline; 1.6891x over previous
import jax
import jax.numpy as jnp
from jax import lax
from jax.experimental import pallas as pl
from jax.experimental.pallas import tpu as pltpu

N_DEV = 4


def _gemm(x, w):
    M, K = x.shape
    _, N = w.shape
    bm, bn = 512, 2048

    def body(x_ref, w_ref, o_ref):
        o_ref[...] = jnp.dot(
            x_ref[...].astype(jnp.bfloat16),
            w_ref[...].astype(jnp.bfloat16),
            preferred_element_type=jnp.float32,
        )

    return pl.pallas_call(
        body,
        grid=(N // bn, M // bm),
        in_specs=[
            pl.BlockSpec((bm, K), lambda n, m: (m, 0)),
            pl.BlockSpec((K, bn), lambda n, m: (0, n)),
        ],
        out_specs=pl.BlockSpec((bm, bn), lambda n, m: (m, n)),
        out_shape=jax.ShapeDtypeStruct((M, N), jnp.float32),
        compiler_params=pltpu.CompilerParams(vmem_limit_bytes=60 * 1024 * 1024),
    )(x, w)


def _rs_silu(partial):
    M, N = partial.shape
    m_out = M // N_DEV
    half = N // 2
    bn = 512
    n_blocks = half // bn
    n_hops = N_DEV - 1

    def body(
        p_ref, o_ref,
        recv_p, recv_m, acc_p, acc_m, lbuf_p, lbuf_m,
        ssem_p, rsem_p, ssem_m, rsem_m,
        lsem_p, lsem_m, osem_p, osem_m, cred_p, cred_m,
    ):
        my = lax.axis_index("i")
        left = lax.rem(my + N_DEV - 1, N_DEV)
        right = lax.rem(my + 1, N_DEV)

        barrier = pltpu.get_barrier_semaphore()
        for nbr in (left, right):
            pl.semaphore_signal(
                barrier, inc=1, device_id=(nbr,),
                device_id_type=pl.DeviceIdType.MESH,
            )
        pl.semaphore_wait(barrier, 2)

        for b in range(n_blocks):
            cols_p = pl.ds(b * bn, bn)
            cols_m = pl.ds(half + b * bn, bn)
            for h in range(n_hops):
                sc_p = lax.rem(my + N_DEV - 1 - h, N_DEV)
                rc_p = lax.rem(my + N_DEV - 2 - h, N_DEV)
                sc_m = lax.rem(my + 1 + h, N_DEV)
                rc_m = lax.rem(my + 2 + h, N_DEV)
                if b * n_hops + h >= n_hops:
                    pl.semaphore_wait(cred_p, 1)
                    pl.semaphore_wait(cred_m, 1)
                src_p = (
                    p_ref.at[pl.ds(sc_p * m_out, m_out), cols_p]
                    if h == 0
                    else acc_p
                )
                src_m = (
                    p_ref.at[pl.ds(sc_m * m_out, m_out), cols_m]
                    if h == 0
                    else acc_m
                )
                rdma_p = pltpu.make_async_remote_copy(
                    src_ref=src_p, dst_ref=recv_p.at[h],
                    send_sem=ssem_p.at[h], recv_sem=rsem_p.at[h],
                    device_id=(right,), device_id_type=pl.DeviceIdType.MESH,
                )
                rdma_m = pltpu.make_async_remote_copy(
                    src_ref=src_m, dst_ref=recv_m.at[h],
                    send_sem=ssem_m.at[h], recv_sem=rsem_m.at[h],
                    device_id=(left,), device_id_type=pl.DeviceIdType.MESH,
                )
                rdma_p.start()
                rdma_m.start()
                lcopy_p = pltpu.make_async_copy(
                    p_ref.at[pl.ds(rc_p * m_out, m_out), cols_p], lbuf_p, lsem_p
                )
                lcopy_m = pltpu.make_async_copy(
                    p_ref.at[pl.ds(rc_m * m_out, m_out), cols_m], lbuf_m, lsem_m
                )
                lcopy_p.start()
                lcopy_m.start()
                rdma_p.wait()
                rdma_m.wait()
                lcopy_p.wait()
                lcopy_m.wait()
                s_p = recv_p[h] + lbuf_p[...]
                s_m = recv_m[h] + lbuf_m[...]
                if h == n_hops - 1:
                    s_p = s_p * jax.nn.sigmoid(s_p)
                    s_m = s_m * jax.nn.sigmoid(s_m)
                acc_p[...] = s_p
                acc_m[...] = s_m
                pl.semaphore_signal(
                    cred_p, inc=1, device_id=(left,),
                    device_id_type=pl.DeviceIdType.MESH,
                )
                pl.semaphore_signal(
                    cred_m, inc=1, device_id=(right,),
                    device_id_type=pl.DeviceIdType.MESH,
                )
            ocopy_p = pltpu.make_async_copy(acc_p, o_ref.at[:, cols_p], osem_p)
            ocopy_m = pltpu.make_async_copy(acc_m, o_ref.at[:, cols_m], osem_m)
            ocopy_p.start()
            ocopy_m.start()
            ocopy_p.wait()
            ocopy_m.wait()
        pl.semaphore_wait(cred_p, n_hops)
        pl.semaphore_wait(cred_m, n_hops)

    return pl.pallas_call(
        body,
        in_specs=[pl.BlockSpec(memory_space=pl.ANY)],
        out_specs=pl.BlockSpec(memory_space=pl.ANY),
        out_shape=jax.ShapeDtypeStruct((m_out, N), jnp.float32),
        scratch_shapes=[
            pltpu.VMEM((n_hops, m_out, bn), jnp.float32),
            pltpu.VMEM((n_hops, m_out, bn), jnp.float32),
            pltpu.VMEM((m_out, bn), jnp.float32),
            pltpu.VMEM((m_out, bn), jnp.float32),
            pltpu.VMEM((m_out, bn), jnp.float32),
            pltpu.VMEM((m_out, bn), jnp.float32),
            pltpu.SemaphoreType.DMA((n_hops,)),
            pltpu.SemaphoreType.DMA((n_hops,)),
            pltpu.SemaphoreType.DMA((n_hops,)),
            pltpu.SemaphoreType.DMA((n_hops,)),
            pltpu.SemaphoreType.DMA,
            pltpu.SemaphoreType.DMA,
            pltpu.SemaphoreType.DMA,
            pltpu.SemaphoreType.DMA,
            pltpu.SemaphoreType.REGULAR,
            pltpu.SemaphoreType.REGULAR,
        ],
        compiler_params=pltpu.CompilerParams(
            collective_id=0, vmem_limit_bytes=60 * 1024 * 1024
        ),
    )(partial)


def kernel(x, w_mat):
    partial = _gemm(x, w_mat)
    return _rs_silu(partial)


# device time: 458941 ns/iter; 2.8792x vs baseline; 1.7047x over previous
import jax
import jax.numpy as jnp
from jax import lax
from jax.experimental import pallas as pl
from jax.experimental.pallas import tpu as pltpu

N_DEV = 4


def _cast_bf16(a, block_rows):
    R, C = a.shape

    def body(a_ref, o_ref):
        o_ref[...] = a_ref[...].astype(jnp.bfloat16)

    return pl.pallas_call(
        body,
        grid=(R // block_rows,),
        in_specs=[pl.BlockSpec((block_rows, C), lambda i: (i, 0))],
        out_specs=pl.BlockSpec((block_rows, C), lambda i: (i, 0)),
        out_shape=jax.ShapeDtypeStruct((R, C), jnp.bfloat16),
    )(a)


def _fused_gemm_rs_silu(xb, wb):
    M, K = xb.shape
    _, N = wb.shape
    m_out = M // N_DEV
    half = N // 2
    bn = 512
    n_blocks = half // bn
    n_hops = N_DEV - 1

    def body(
        x_ref, w_ref, o_ref,
        recv_p, recv_m, sbuf_p, sbuf_m, xbuf_p, xbuf_m,
        wbuf_p, wbuf_m, gbuf_p, gbuf_m, obuf_p, obuf_m,
        ssem_p, rsem_p, ssem_m, rsem_m,
        xsem_p, xsem_m, wsem_p, wsem_m, osem_p, osem_m,
        cred_p, cred_m,
    ):
        my = lax.axis_index("i")
        left = lax.rem(my + N_DEV - 1, N_DEV)
        right = lax.rem(my + 1, N_DEV)

        barrier = pltpu.get_barrier_semaphore()
        for nbr in (left, right):
            pl.semaphore_signal(
                barrier, inc=1, device_id=(nbr,),
                device_id_type=pl.DeviceIdType.MESH,
            )
        pl.semaphore_wait(barrier, 2)

        def dot(a, b):
            return jnp.dot(a, b, preferred_element_type=jnp.float32)

        def pass_body(b, carry):
            cols_p = pl.ds(b * bn, bn)
            cols_m = pl.ds(half + b * bn, bn)

            wcopy_p = pltpu.make_async_copy(w_ref.at[:, cols_p], wbuf_p, wsem_p)
            wcopy_m = pltpu.make_async_copy(w_ref.at[:, cols_m], wbuf_m, wsem_m)
            wcopy_p.start()
            wcopy_m.start()
            sc_p0 = lax.rem(my + N_DEV - 1, N_DEV)
            sc_m0 = lax.rem(my + 1, N_DEV)
            xcopy_p = pltpu.make_async_copy(
                x_ref.at[pl.ds(sc_p0 * m_out, m_out), :], xbuf_p, xsem_p
            )
            xcopy_m = pltpu.make_async_copy(
                x_ref.at[pl.ds(sc_m0 * m_out, m_out), :], xbuf_m, xsem_m
            )
            xcopy_p.start()
            xcopy_m.start()
            wcopy_p.wait()
            wcopy_m.wait()
            xcopy_p.wait()
            xcopy_m.wait()
            sbuf_p[0, :, :] = dot(xbuf_p[...], wbuf_p[...]).astype(jnp.bfloat16)
            sbuf_m[0, :, :] = dot(xbuf_m[...], wbuf_m[...]).astype(jnp.bfloat16)

            for h in range(n_hops):
                cur = h % 2
                nxt = 1 - cur
                rc_p = lax.rem(my + N_DEV - 2 - h, N_DEV)
                rc_m = lax.rem(my + 2 + h, N_DEV)

                @pl.when(b >= 1)
                def _():
                    pl.semaphore_wait(cred_p, 1)
                    pl.semaphore_wait(cred_m, 1)
                rdma_p = pltpu.make_async_remote_copy(
                    src_ref=sbuf_p.at[cur], dst_ref=recv_p.at[h],
                    send_sem=ssem_p.at[h], recv_sem=rsem_p.at[h],
                    device_id=(right,), device_id_type=pl.DeviceIdType.MESH,
                )
                rdma_m = pltpu.make_async_remote_copy(
                    src_ref=sbuf_m.at[cur], dst_ref=recv_m.at[h],
                    send_sem=ssem_m.at[h], recv_sem=rsem_m.at[h],
                    device_id=(left,), device_id_type=pl.DeviceIdType.MESH,
                )
                rdma_p.start()
                rdma_m.start()
                xcopy_p = pltpu.make_async_copy(
                    x_ref.at[pl.ds(rc_p * m_out, m_out), :], xbuf_p, xsem_p
                )
                xcopy_m = pltpu.make_async_copy(
                    x_ref.at[pl.ds(rc_m * m_out, m_out), :], xbuf_m, xsem_m
                )
                xcopy_p.start()
                xcopy_m.start()
                xcopy_p.wait()
                xcopy_m.wait()
                gbuf_p[...] = dot(xbuf_p[...], wbuf_p[...])
                gbuf_m[...] = dot(xbuf_m[...], wbuf_m[...])
                rdma_p.wait()
                rdma_m.wait()
                s_p = gbuf_p[...] + recv_p[h].astype(jnp.float32)
                s_m = gbuf_m[...] + recv_m[h].astype(jnp.float32)
                if h == n_hops - 1:
                    obuf_p[...] = s_p * jax.nn.sigmoid(s_p)
                    obuf_m[...] = s_m * jax.nn.sigmoid(s_m)
                else:
                    sbuf_p[nxt, :, :] = s_p.astype(jnp.bfloat16)
                    sbuf_m[nxt, :, :] = s_m.astype(jnp.bfloat16)
                pl.semaphore_signal(
                    cred_p, inc=1, device_id=(left,),
                    device_id_type=pl.DeviceIdType.MESH,
                )
                pl.semaphore_signal(
                    cred_m, inc=1, device_id=(right,),
                    device_id_type=pl.DeviceIdType.MESH,
                )
            ocopy_p = pltpu.make_async_copy(obuf_p, o_ref.at[:, cols_p], osem_p)
            ocopy_m = pltpu.make_async_copy(obuf_m, o_ref.at[:, cols_m], osem_m)
            ocopy_p.start()
            ocopy_m.start()
            ocopy_p.wait()
            ocopy_m.wait()
            return carry

        lax.fori_loop(0, n_blocks, pass_body, 0)
        pl.semaphore_wait(cred_p, n_hops)
        pl.semaphore_wait(cred_m, n_hops)

    return pl.pallas_call(
        body,
        in_specs=[
            pl.BlockSpec(memory_space=pl.ANY),
            pl.BlockSpec(memory_space=pl.ANY),
        ],
        out_specs=pl.BlockSpec(memory_space=pl.ANY),
        out_shape=jax.ShapeDtypeStruct((m_out, N), jnp.float32),
        scratch_shapes=[
            pltpu.VMEM((n_hops, m_out, bn), jnp.bfloat16),
            pltpu.VMEM((n_hops, m_out, bn), jnp.bfloat16),
            pltpu.VMEM((2, m_out, bn), jnp.bfloat16),
            pltpu.VMEM((2, m_out, bn), jnp.bfloat16),
            pltpu.VMEM((m_out, K), jnp.bfloat16),
            pltpu.VMEM((m_out, K), jnp.bfloat16),
            pltpu.VMEM((K, bn), jnp.bfloat16),
            pltpu.VMEM((K, bn), jnp.bfloat16),
            pltpu.VMEM((m_out, bn), jnp.float32),
            pltpu.VMEM((m_out, bn), jnp.float32),
            pltpu.VMEM((m_out, bn), jnp.float32),
            pltpu.VMEM((m_out, bn), jnp.float32),
            pltpu.SemaphoreType.DMA((n_hops,)),
            pltpu.SemaphoreType.DMA((n_hops,)),
            pltpu.SemaphoreType.DMA((n_hops,)),
            pltpu.SemaphoreType.DMA((n_hops,)),
            pltpu.SemaphoreType.DMA,
            pltpu.SemaphoreType.DMA,
            pltpu.SemaphoreType.DMA,
            pltpu.SemaphoreType.DMA,
            pltpu.SemaphoreType.DMA,
            pltpu.SemaphoreType.DMA,
            pltpu.SemaphoreType.REGULAR,
            pltpu.SemaphoreType.REGULAR,
        ],
        compiler_params=pltpu.CompilerParams(
            collective_id=0, vmem_limit_bytes=62 * 1024 * 1024
        ),
    )(xb, wb)


def kernel(x, w_mat):
    xb = _cast_bf16(x, 1024)
    wb = _cast_bf16(w_mat, 512)
    return _fused_gemm_rs_silu(xb, wb)
